# baseline (speedup 1.0000x reference)
import functools

import jax
import jax.numpy as jnp
from jax import lax
from jax.experimental import pallas as pl
from jax.experimental.pallas import tpu as pltpu
from jax.experimental.pallas import tpu_sc as plsc

N_NODES = 100000
N_FEATS = 9
VOCAB = 512
DIM = 128
DP = DIM // 2
LANES = 16

NB = 160
NCHUNKS = N_NODES // NB
NW = 32
NGROUPS = NB // LANES
IDX_W = NB * N_FEATS
OUT_W = NB * DIM

_mesh = plsc.VectorSubcoreMesh(core_axis_name="c", subcore_axis_name="s")


@functools.partial(
    pl.kernel,
    out_type=jax.ShapeDtypeStruct((N_NODES * DIM,), jnp.float32),
    mesh=_mesh,
    compiler_params=pltpu.CompilerParams(needs_layout_passes=False),
    scratch_types=[
        pltpu.VMEM((VOCAB * DP,), jnp.int32),
        pltpu.VMEM((IDX_W,), jnp.int32),
        pltpu.VMEM((IDX_W,), jnp.int32),
        pltpu.VMEM((OUT_W,), jnp.float32),
        pltpu.VMEM((OUT_W,), jnp.float32),
        pltpu.SemaphoreType.DMA,
        pltpu.SemaphoreType.DMA,
        pltpu.SemaphoreType.DMA,
    ],
)
def _embed_sum(
    nf_hbm, tab_hbm, out_hbm,
    tab_v, idx_v0, idx_v1, out_v0, out_v1,
    sem_i1, sem_o0, sem_o1,
):
    wid = lax.axis_index("s") * 2 + lax.axis_index("c")
    pltpu.sync_copy(tab_hbm, tab_v)

    lane = lax.iota(jnp.int32, LANES)
    lane9 = lane * N_FEATS
    laneDIM = lane * DIM

    def process(chunk, idx_v, out_v, sem_o):
        for g in range(NGROUPS):
            rowsc = []
            for f in range(N_FEATS):
                rows = plsc.load_gather(idx_v, [lane9 + (g * LANES * N_FEATS + f)])
                rowsc.append(rows * DP)

            @plsc.parallel_loop(0, DP, unroll=4)
            def do_dim(dp, rowsc=rowsc, g=g):
                dp_vec = (dp & ~(LANES - 1)) + ((lane + dp) & (LANES - 1))
                acc = plsc.bitcast(
                    plsc.load_gather(tab_v, [rowsc[0] + dp_vec]), jnp.bfloat16
                )
                for f in range(1, N_FEATS):
                    acc = acc + plsc.bitcast(
                        plsc.load_gather(tab_v, [rowsc[f] + dp_vec]),
                        jnp.bfloat16,
                    )
                pk = plsc.bitcast(acc, jnp.int32)
                lo = plsc.bitcast(lax.shift_left(pk, 16), jnp.float32)
                hi = plsc.bitcast(pk & jnp.int32(-65536), jnp.float32)
                out_lo = laneDIM + g * LANES * DIM + dp_vec
                plsc.store_scatter(out_v, [out_lo], lo)
                plsc.store_scatter(out_v, [out_lo + DP], hi)

        pltpu.async_copy(out_v, out_hbm.at[pl.ds(chunk * OUT_W, OUT_W)], sem_o)

    def wait_out(out_v, sem_o):
        pltpu.make_async_copy(out_hbm.at[pl.ds(0, OUT_W)], out_v, sem_o).wait()

    n_w = (NCHUNKS - wid + NW - 1) // NW

    def do_pair(jj, carry):
        chunk0 = wid + (2 * jj) * NW
        chunk1 = chunk0 + NW
        have1 = 2 * jj + 1 < n_w

        pltpu.sync_copy(nf_hbm.at[pl.ds(chunk0 * IDX_W, IDX_W)], idx_v0)

        @pl.when(have1)
        def _():
            pltpu.async_copy(
                nf_hbm.at[pl.ds(chunk1 * IDX_W, IDX_W)], idx_v1, sem_i1
            )

        @pl.when(jj > 0)
        def _():
            wait_out(out_v0, sem_o0)

        process(chunk0, idx_v0, out_v0, sem_o0)

        @pl.when(have1)
        def _():
            pltpu.make_async_copy(
                nf_hbm.at[pl.ds(chunk1 * IDX_W, IDX_W)], idx_v1, sem_i1
            ).wait()

            @pl.when(jj > 0)
            def _():
                wait_out(out_v1, sem_o1)

            process(chunk1, idx_v1, out_v1, sem_o1)

        return carry

    lax.fori_loop(0, (n_w + 1) // 2, do_pair, 0)
    wait_out(out_v0, sem_o0)
    wait_out(out_v1, sem_o1)


def kernel(node_features, codebook_weight):
    nf_flat = node_features.astype(jnp.int32).reshape(-1)
    tab_bf = codebook_weight.astype(jnp.bfloat16)
    tab_pairs = jnp.stack([tab_bf[:, :DP], tab_bf[:, DP:]], axis=-1)
    tab_pk = jax.lax.bitcast_convert_type(tab_pairs, jnp.int32).reshape(-1)
    out = _embed_sum(nf_flat, tab_pk)
    return out.reshape(N_NODES, DIM)


# speedup vs baseline: 1.0871x; 1.0871x over previous
import functools

import jax
import jax.numpy as jnp
from jax import lax
from jax.experimental import pallas as pl
from jax.experimental.pallas import tpu as pltpu
from jax.experimental.pallas import tpu_sc as plsc

N_NODES = 100000
N_FEATS = 9
VOCAB = 512
DIM = 128
DP = DIM // 2
LANES = 16

NB = 160
NCHUNKS = N_NODES // NB
NW = 32
NGROUPS = NB // LANES
IDX_W = NB * N_FEATS
OUT_W = NB * DIM

_mesh = plsc.VectorSubcoreMesh(core_axis_name="c", subcore_axis_name="s")


@functools.partial(
    pl.kernel,
    out_type=jax.ShapeDtypeStruct((N_NODES * DIM,), jnp.float32),
    mesh=_mesh,
    compiler_params=pltpu.CompilerParams(needs_layout_passes=False),
    scratch_types=[
        pltpu.VMEM((VOCAB * DP,), jnp.int32),
        pltpu.VMEM((IDX_W,), jnp.int32),
        pltpu.VMEM((IDX_W,), jnp.int32),
        pltpu.VMEM((OUT_W,), jnp.float32),
        pltpu.VMEM((OUT_W,), jnp.float32),
        pltpu.SemaphoreType.DMA,
        pltpu.SemaphoreType.DMA,
        pltpu.SemaphoreType.DMA,
    ],
)
def _embed_sum(
    nf_hbm, tab_hbm, out_hbm,
    tab_v, idx_v0, idx_v1, out_v0, out_v1,
    sem_i1, sem_o0, sem_o1,
):
    wid = lax.axis_index("s") * 2 + lax.axis_index("c")
    pltpu.sync_copy(tab_hbm, tab_v)

    lane = lax.iota(jnp.int32, LANES)
    lane9 = lane * N_FEATS
    laneDIM = lane * DIM

    def process(chunk, idx_v, out_v, sem_o):
        for g in range(NGROUPS):
            rowsc = []
            for f in range(N_FEATS):
                rows = plsc.load_gather(idx_v, [lane9 + (g * LANES * N_FEATS + f)])
                rowsc.append(rows * DP)

            @plsc.parallel_loop(0, DP, unroll=1)
            def do_dim(dp, rowsc=rowsc, g=g):
                dp_vec = (dp & ~(LANES - 1)) + ((lane + dp) & (LANES - 1))
                acc = plsc.bitcast(
                    plsc.load_gather(tab_v, [rowsc[0] + dp_vec]), jnp.bfloat16
                )
                for f in range(1, N_FEATS):
                    acc = acc + plsc.bitcast(
                        plsc.load_gather(tab_v, [rowsc[f] + dp_vec]),
                        jnp.bfloat16,
                    )
                pk = plsc.bitcast(acc, jnp.int32)
                lo = plsc.bitcast(lax.shift_left(pk, 16), jnp.float32)
                hi = plsc.bitcast(pk & jnp.int32(-65536), jnp.float32)
                out_lo = laneDIM + g * LANES * DIM + dp_vec
                plsc.store_scatter(out_v, [out_lo], lo)
                plsc.store_scatter(out_v, [out_lo + DP], hi)

        pltpu.async_copy(out_v, out_hbm.at[pl.ds(chunk * OUT_W, OUT_W)], sem_o)

    def wait_out(out_v, sem_o):
        pltpu.make_async_copy(out_hbm.at[pl.ds(0, OUT_W)], out_v, sem_o).wait()

    n_w = (NCHUNKS - wid + NW - 1) // NW

    def do_pair(jj, carry):
        chunk0 = wid + (2 * jj) * NW
        chunk1 = chunk0 + NW
        have1 = 2 * jj + 1 < n_w

        pltpu.sync_copy(nf_hbm.at[pl.ds(chunk0 * IDX_W, IDX_W)], idx_v0)

        @pl.when(have1)
        def _():
            pltpu.async_copy(
                nf_hbm.at[pl.ds(chunk1 * IDX_W, IDX_W)], idx_v1, sem_i1
            )

        @pl.when(jj > 0)
        def _():
            wait_out(out_v0, sem_o0)

        process(chunk0, idx_v0, out_v0, sem_o0)

        @pl.when(have1)
        def _():
            pltpu.make_async_copy(
                nf_hbm.at[pl.ds(chunk1 * IDX_W, IDX_W)], idx_v1, sem_i1
            ).wait()

            @pl.when(jj > 0)
            def _():
                wait_out(out_v1, sem_o1)

            process(chunk1, idx_v1, out_v1, sem_o1)

        return carry

    lax.fori_loop(0, (n_w + 1) // 2, do_pair, 0)
    wait_out(out_v0, sem_o0)
    wait_out(out_v1, sem_o1)


def kernel(node_features, codebook_weight):
    nf_flat = node_features.astype(jnp.int32).reshape(-1)
    tab_bf = codebook_weight.astype(jnp.bfloat16)
    tab_pairs = jnp.stack([tab_bf[:, :DP], tab_bf[:, DP:]], axis=-1)
    tab_pk = jax.lax.bitcast_convert_type(tab_pairs, jnp.int32).reshape(-1)
    out = _embed_sum(nf_flat, tab_pk)
    return out.reshape(N_NODES, DIM)
